# baseline (device time: 66248 ns/iter reference)
import numpy as np

import jax
import jax.numpy as jnp
from jax import lax
from jax.experimental import pallas as pl
from jax.experimental.pallas import tpu as pltpu

N_DEV = 4
_BLOCK_MIN_J = 1


def _blocked_stage(val, j, k, base, force_asc=False):
    length, n = val.shape
    nb = length // (2 * j)
    v = val.reshape(nb, 2, j, n)
    a, b = v[:, 0], v[:, 1]
    lo, hi = jnp.minimum(a, b), jnp.maximum(a, b)
    static_dirs = (
        ((np.arange(nb) * 2 * j + base) & k) == 0
        if isinstance(base, int)
        else None
    )
    if force_asc or (static_dirs is not None and static_dirs.all()):
        first, second = lo, hi
    elif static_dirs is not None and not static_dirs.any():
        first, second = hi, lo
    else:
        bid = lax.broadcasted_iota(jnp.int32, (nb, 1, 1), 0)
        ab = ((bid * (2 * j) + base) & k) == 0
        first = jnp.where(ab, lo, hi)
        second = jnp.where(ab, hi, lo)
    return jnp.stack([first, second], axis=1).reshape(length, n)


def _roll_stages(val, j_start, k, base, force_asc=False):
    length, _ = val.shape
    row1 = lax.broadcasted_iota(jnp.int32, (length, 1), 0)
    asc = True if force_asc else ((row1 + base) & k) == 0
    j = j_start
    while j >= 1:
        down = jnp.concatenate([val[j:], val[:j]], axis=0)
        up = jnp.concatenate([val[length - j:], val[:length - j]], axis=0)
        upper = (row1 & j) != 0
        pval = jnp.where(upper, up, down)
        take_min = upper != asc
        val = jnp.where(take_min, jnp.minimum(val, pval), jnp.maximum(val, pval))
        j //= 2
    return val


def _bitonic_merge(val, k, base):
    j = k // 2
    while j >= _BLOCK_MIN_J:
        val = _blocked_stage(val, j, k, base)
        j //= 2
    if j >= 1:
        val = _roll_stages(val, j, k, base)
    return val


def kernel(x):
    m_per, n = x.shape
    m_total = N_DEV * m_per

    def body(x_ref, out_ref, gath_ref, send_sems, recv_sems):
        my = lax.axis_index("i")
        left = (my - 1) % N_DEV
        right = (my + 1) % N_DEV

        val = x_ref[...].astype(jnp.bfloat16)
        k = 2
        while k <= m_per:
            base = 0 if k < m_per else my * m_per
            val = _bitonic_merge(val, k, base)
            k *= 2
        gath_ref[pl.ds(my * m_per, m_per), :] = val

        barrier_sem = pltpu.get_barrier_semaphore()
        for nbr in [left, right]:
            pl.semaphore_signal(
                barrier_sem, inc=1,
                device_id=(nbr,), device_id_type=pl.DeviceIdType.MESH,
            )
        pl.semaphore_wait(barrier_sem, 2)

        def copy(slot, sem_idx, target, row0=0, nrows=m_per):
            return pltpu.make_async_remote_copy(
                src_ref=gath_ref.at[pl.ds(slot * m_per + row0, nrows)],
                dst_ref=gath_ref.at[pl.ds(slot * m_per + row0, nrows)],
                send_sem=send_sems.at[sem_idx],
                recv_sem=recv_sems.at[sem_idx],
                device_id=(target,),
                device_id_type=pl.DeviceIdType.MESH,
            )

        send_r = copy(my, 0, right)
        send_r.start()
        send_l = copy(my, 1, left)
        send_l.start()

        recv_l = copy(left, 0, left)
        recv_l.wait_recv()
        hhalf = m_per // 2
        fwd_r = copy(left, 2, right, row0=0, nrows=hhalf)
        fwd_r.start()
        recv_r = copy(right, 1, right)
        recv_r.wait_recv()
        fwd_l = copy(right, 3, left, row0=hhalf, nrows=hhalf)
        fwd_l.start()

        p0 = my - (my % 2)
        q0 = (p0 + 2) % N_DEV
        half = 2 * m_per
        own = gath_ref[pl.ds(p0 * m_per, half), :]
        own = _bitonic_merge(own, half, p0 * m_per)

        diag = (my - 2) % N_DEV
        copy(diag, 2, left, row0=0, nrows=hhalf).wait_recv()
        copy(diag, 3, right, row0=hhalf, nrows=hhalf).wait_recv()
        other = gath_ref[pl.ds(q0 * m_per, half), :]
        other = _bitonic_merge(other, half, q0 * m_per)

        own_asc = p0 < 2
        block_a = jnp.where(own_asc, own, other)
        block_b = jnp.where(own_asc, other, own)
        val = jnp.concatenate([block_a, block_b], axis=0)
        j = m_total // 2
        while j >= 32:
            val = _blocked_stage(val, j, m_total, 0)
            j //= 2
        gath_ref[...] = val

        win = m_per + 64
        start = pl.multiple_of(jnp.clip(my * m_per - 32, 0, m_total - win), 32)
        wval = gath_ref[pl.ds(start, win), :]
        j = 16
        while j >= _BLOCK_MIN_J:
            wval = _blocked_stage(wval, j, m_total, 0, force_asc=True)
            j //= 2
        if j >= 1:
            wval = _roll_stages(wval, j, m_total, 0, force_asc=True)
        gath_ref[pl.ds(start, win), :] = wval

        out_ref[...] = gath_ref[pl.ds(my * m_per, m_per), :].astype(jnp.float32)

        send_r.wait_send()
        send_l.wait_send()
        fwd_r.wait_send()
        fwd_l.wait_send()

    return pl.pallas_call(
        body,
        out_shape=jax.ShapeDtypeStruct((m_per, n), jnp.float32),
        in_specs=[pl.BlockSpec(memory_space=pltpu.VMEM)],
        out_specs=pl.BlockSpec(memory_space=pltpu.VMEM),
        scratch_shapes=[
            pltpu.VMEM((m_total, n), jnp.bfloat16),
            pltpu.SemaphoreType.DMA((4,)),
            pltpu.SemaphoreType.DMA((4,)),
        ],
        compiler_params=pltpu.CompilerParams(collective_id=0),
    )(x)


# device time: 34921 ns/iter; 1.8971x vs baseline; 1.8971x over previous
import numpy as np

import jax
import jax.numpy as jnp
from jax import lax
from jax.experimental import pallas as pl
from jax.experimental.pallas import tpu as pltpu

N_DEV = 4
_BLOCK_MIN_J = 16


def _blocked_stage(val, j, k, base, force_asc=False):
    length, n = val.shape
    nb = length // (2 * j)
    v = val.reshape(nb, 2, j, n)
    a, b = v[:, 0], v[:, 1]
    lo, hi = jnp.minimum(a, b), jnp.maximum(a, b)
    static_dirs = (
        ((np.arange(nb) * 2 * j + base) & k) == 0
        if isinstance(base, int)
        else None
    )
    if force_asc or (static_dirs is not None and static_dirs.all()):
        first, second = lo, hi
    elif static_dirs is not None and not static_dirs.any():
        first, second = hi, lo
    else:
        bid = lax.broadcasted_iota(jnp.int32, (nb, 1, 1), 0)
        ab = ((bid * (2 * j) + base) & k) == 0
        first = jnp.where(ab, lo, hi)
        second = jnp.where(ab, hi, lo)
    return jnp.stack([first, second], axis=1).reshape(length, n)


def _roll_stages(val, j_start, k, base, force_asc=False):
    length, _ = val.shape
    row1 = lax.broadcasted_iota(jnp.int32, (length, 1), 0)
    asc = True if force_asc else ((row1 + base) & k) == 0
    j = j_start
    while j >= 1:
        down = jnp.concatenate([val[j:], val[:j]], axis=0)
        up = jnp.concatenate([val[length - j:], val[:length - j]], axis=0)
        upper = (row1 & j) != 0
        pval = jnp.where(upper, up, down)
        take_min = upper != asc
        val = jnp.where(take_min, jnp.minimum(val, pval), jnp.maximum(val, pval))
        j //= 2
    return val


def _bitonic_merge(val, k, base):
    j = k // 2
    while j >= _BLOCK_MIN_J:
        val = _blocked_stage(val, j, k, base)
        j //= 2
    if j >= 1:
        val = _roll_stages(val, j, k, base)
    return val


def kernel(x):
    m_per, n = x.shape
    m_total = N_DEV * m_per

    def body(x_ref, out_ref, gath_ref, send_sems, recv_sems):
        my = lax.axis_index("i")
        left = (my - 1) % N_DEV
        right = (my + 1) % N_DEV

        barrier_sem = pltpu.get_barrier_semaphore()
        for nbr in [left, right]:
            pl.semaphore_signal(
                barrier_sem, inc=1,
                device_id=(nbr,), device_id_type=pl.DeviceIdType.MESH,
            )

        val = x_ref[...].astype(jnp.bfloat16)
        k = 2
        while k <= m_per:
            base = 0 if k < m_per else my * m_per
            val = _bitonic_merge(val, k, base)
            k *= 2
        gath_ref[pl.ds(my * m_per, m_per), :] = val

        pl.semaphore_wait(barrier_sem, 2)

        def copy(slot, sem_idx, target, row0=0, nrows=m_per):
            return pltpu.make_async_remote_copy(
                src_ref=gath_ref.at[pl.ds(slot * m_per + row0, nrows)],
                dst_ref=gath_ref.at[pl.ds(slot * m_per + row0, nrows)],
                send_sem=send_sems.at[sem_idx],
                recv_sem=recv_sems.at[sem_idx],
                device_id=(target,),
                device_id_type=pl.DeviceIdType.MESH,
            )

        send_r = copy(my, 0, right)
        send_r.start()
        send_l = copy(my, 1, left)
        send_l.start()

        recv_l = copy(left, 0, left)
        recv_l.wait_recv()
        hhalf = m_per // 2
        fwd_r = copy(left, 2, right, row0=0, nrows=hhalf)
        fwd_r.start()
        recv_r = copy(right, 1, right)
        recv_r.wait_recv()
        fwd_l = copy(right, 3, left, row0=hhalf, nrows=hhalf)
        fwd_l.start()

        p0 = my - (my % 2)
        q0 = (p0 + 2) % N_DEV
        half = 2 * m_per
        own = gath_ref[pl.ds(p0 * m_per, half), :]
        own = _bitonic_merge(own, half, p0 * m_per)

        diag = (my - 2) % N_DEV
        copy(diag, 2, left, row0=0, nrows=hhalf).wait_recv()
        copy(diag, 3, right, row0=hhalf, nrows=hhalf).wait_recv()
        other = gath_ref[pl.ds(q0 * m_per, half), :]
        other = _bitonic_merge(other, half, q0 * m_per)

        own_asc = p0 < 2
        block_a = jnp.where(own_asc, own, other)
        block_b = jnp.where(own_asc, other, own)
        val = jnp.concatenate([block_a, block_b], axis=0)
        j = m_total // 2
        while j >= 32:
            val = _blocked_stage(val, j, m_total, 0)
            j //= 2
        gath_ref[...] = val

        win = m_per + 64
        start = pl.multiple_of(jnp.clip(my * m_per - 32, 0, m_total - win), 32)
        wval = gath_ref[pl.ds(start, win), :]
        j = 16
        while j >= _BLOCK_MIN_J:
            wval = _blocked_stage(wval, j, m_total, 0, force_asc=True)
            j //= 2
        if j >= 1:
            wval = _roll_stages(wval, j, m_total, 0, force_asc=True)
        gath_ref[pl.ds(start, win), :] = wval

        out_ref[...] = gath_ref[pl.ds(my * m_per, m_per), :]

        send_r.wait_send()
        send_l.wait_send()
        fwd_r.wait_send()
        fwd_l.wait_send()

    return pl.pallas_call(
        body,
        out_shape=jax.ShapeDtypeStruct((m_per, n), jnp.bfloat16),
        in_specs=[pl.BlockSpec(memory_space=pltpu.VMEM)],
        out_specs=pl.BlockSpec(memory_space=pltpu.VMEM),
        scratch_shapes=[
            pltpu.VMEM((m_total, n), jnp.bfloat16),
            pltpu.SemaphoreType.DMA((4,)),
            pltpu.SemaphoreType.DMA((4,)),
        ],
        compiler_params=pltpu.CompilerParams(collective_id=0),
    )(x)
